# baseline (speedup 1.0000x reference)
import functools

import jax
import jax.numpy as jnp
from jax import lax
from jax.experimental import pallas as pl
from jax.experimental.pallas import tpu as pltpu
from jax.experimental.pallas import tpu_sc as plsc

N = 100000
D = 128
NUM_CLASSES = 1000

B = 200
G = 128
T = 16
NW = 32
L = 16
SPAN = B * T
W_FULL = 20


def _mask_label_sc(x_hbm, y_hbm, m_hbm, tab_hbm, out_hbm,
                   yv, mv, idxf, xa, xb, ra, rb, tab_sp,
                   sem_xa, sem_xb, sem_ga, sem_gb, sem_oa, sem_ob, sem_t):
    sid = lax.axis_index("s")
    wid = sid * 2 + lax.axis_index("c")
    cnt = jnp.where(wid < W_FULL, T, T - 1)
    rbase = 3000 * wid + 200 * jnp.minimum(wid, W_FULL)

    bufs = ((xa, ra, sem_xa, sem_ga), (xb, rb, sem_xb, sem_gb))
    out_sems = (sem_oa, sem_ob)

    def chunk_base(t):
        ct = jnp.minimum(jnp.int32(t), cnt - 1)
        return ct, rbase + B * ct

    def issue_x(t):
        xv = bufs[t % 2][0]
        _, base = chunk_base(t)
        return pltpu.async_copy(x_hbm.at[pl.ds(base, B)], xv, bufs[t % 2][2])

    def issue_g(t):
        rv, sg = bufs[t % 2][1], bufs[t % 2][3]
        ct, _ = chunk_base(t)
        ib = ct * B
        h0 = pltpu.async_copy(tab_sp.at[idxf.at[pl.ds(ib, G)]],
                              rv.at[pl.ds(0, G)], sg)
        h1 = pltpu.async_copy(tab_sp.at[idxf.at[pl.ds(ib + G, B - G)]],
                              rv.at[pl.ds(G, B - G)], sg)
        return h0, h1

    h_tab = pltpu.async_copy(tab_hbm.at[pl.ds(64 * sid, 64)],
                             tab_sp.at[pl.ds(64 * sid, 64)], sem_t)
    xh = {0: issue_x(0), 1: issue_x(1)}
    hy = pltpu.async_copy(y_hbm.at[pl.ds(rbase, 3000)],
                          yv.at[pl.ds(0, 3000)], sem_oa)
    hm = pltpu.async_copy(m_hbm.at[pl.ds(rbase, 3000)],
                          mv.at[pl.ds(0, 3000)], sem_ob)
    hy.wait()
    hm.wait()

    @pl.when(wid < W_FULL)
    def _():
        pltpu.sync_copy(y_hbm.at[pl.ds(rbase + 3000, 200)],
                        yv.at[pl.ds(3000, 200)])
        pltpu.sync_copy(m_hbm.at[pl.ds(rbase + 3000, 200)],
                        mv.at[pl.ds(3000, 200)])

    @plsc.parallel_loop(0, SPAN // L, unroll=2)
    def _(j):
        sl = pl.ds(j * L, L)
        idxf[sl] = jnp.where(mv[sl] != 0, yv[sl], jnp.int32(NUM_CLASSES))

    h_tab.wait()
    plsc.subcore_barrier()

    gh = {0: issue_g(0)}
    out_flight = {}
    for t in range(T):
        xv, rv, _, _ = bufs[t % 2]
        if t + 1 < T:
            if t >= 1:
                out_flight.pop(t - 1).wait()
                xh[t + 1] = issue_x(t + 1)
            gh[t + 1] = issue_g(t + 1)
        xh.pop(t).wait()
        h0, h1 = gh.pop(t)
        h0.wait()

        @plsc.parallel_loop(0, G, unroll=4)
        def _(r):
            for cc in range(D // L):
                sl = pl.ds(cc * L, L)
                xv[r, sl] = xv[r, sl] + rv[r, sl]

        h1.wait()

        @plsc.parallel_loop(G, B, unroll=4)
        def _(r):
            for cc in range(D // L):
                sl = pl.ds(cc * L, L)
                xv[r, sl] = xv[r, sl] + rv[r, sl]

        _, base = chunk_base(t)
        out_flight[t] = pltpu.async_copy(xv, out_hbm.at[pl.ds(base, B)],
                                         out_sems[t % 2])
    out_flight.pop(T - 2).wait()
    out_flight.pop(T - 1).wait()


@jax.jit
def _run(x, y, m_i32, table):
    mesh = plsc.VectorSubcoreMesh(core_axis_name="c", subcore_axis_name="s")
    f = functools.partial(
        pl.kernel,
        out_type=jax.ShapeDtypeStruct((N, D), jnp.float32),
        mesh=mesh,
        scratch_types=[
            pltpu.VMEM((SPAN,), jnp.int32),
            pltpu.VMEM((SPAN,), jnp.int32),
            pltpu.VMEM((SPAN,), jnp.int32),
            pltpu.VMEM((B, D), jnp.float32),
            pltpu.VMEM((B, D), jnp.float32),
            pltpu.VMEM((B, D), jnp.float32),
            pltpu.VMEM((B, D), jnp.float32),
            pltpu.VMEM_SHARED((1024, D), jnp.float32),
            pltpu.SemaphoreType.DMA,
            pltpu.SemaphoreType.DMA,
            pltpu.SemaphoreType.DMA,
            pltpu.SemaphoreType.DMA,
            pltpu.SemaphoreType.DMA,
            pltpu.SemaphoreType.DMA,
            pltpu.SemaphoreType.DMA,
        ],
    )(_mask_label_sc)
    return f(x, y, m_i32, table)


def kernel(x, y, mask, emb_weight):
    m_i32 = mask.astype(jnp.int32)
    table = jnp.concatenate(
        [emb_weight, jnp.zeros((24, D), jnp.float32)], axis=0)
    return _run(x, y, m_i32, table)


# speedup vs baseline: 1.0465x; 1.0465x over previous
import functools

import jax
import jax.numpy as jnp
from jax import lax
from jax.experimental import pallas as pl
from jax.experimental.pallas import tpu as pltpu
from jax.experimental.pallas import tpu_sc as plsc

N = 100000
D = 128
NUM_CLASSES = 1000

B = 200
G = 128
T = 16
NW = 32
L = 16
SPAN = B * T
W_FULL = 20


def _mask_label_sc(x_hbm, y_hbm, m_hbm, tab_hbm, out_hbm,
                   yv, mv, idxf, xa, xb, ra, rb, tab_sp,
                   sem_xa, sem_xb, sem_ga, sem_gb, sem_oa, sem_ob, sem_t):
    sid = lax.axis_index("s")
    wid = sid * 2 + lax.axis_index("c")
    cnt = jnp.where(wid < W_FULL, T, T - 1)
    rbase = 3000 * wid + 200 * jnp.minimum(wid, W_FULL)

    bufs = ((xa, ra, sem_xa, sem_ga), (xb, rb, sem_xb, sem_gb))
    out_sems = (sem_oa, sem_ob)

    def chunk_base(t):
        ct = jnp.minimum(jnp.int32(t), cnt - 1)
        return ct, rbase + B * ct

    def issue_x(t):
        xv = bufs[t % 2][0]
        _, base = chunk_base(t)
        return pltpu.async_copy(x_hbm.at[pl.ds(base, B)], xv, bufs[t % 2][2])

    def issue_g(t):
        rv, sg = bufs[t % 2][1], bufs[t % 2][3]
        ct, _ = chunk_base(t)
        ib = ct * B
        h0 = pltpu.async_copy(tab_sp.at[idxf.at[pl.ds(ib, G)]],
                              rv.at[pl.ds(0, G)], sg)
        h1 = pltpu.async_copy(tab_sp.at[idxf.at[pl.ds(ib + G, B - G)]],
                              rv.at[pl.ds(G, B - G)], sg)
        return h0, h1

    h_tab = pltpu.async_copy(tab_hbm.at[pl.ds(64 * sid, 64)],
                             tab_sp.at[pl.ds(64 * sid, 64)], sem_t)
    xh = {0: issue_x(0), 1: issue_x(1)}
    hy = pltpu.async_copy(y_hbm.at[pl.ds(rbase, 3000)],
                          yv.at[pl.ds(0, 3000)], sem_oa)
    hm = pltpu.async_copy(m_hbm.at[pl.ds(rbase, 3000)],
                          mv.at[pl.ds(0, 3000)], sem_ob)
    hy.wait()
    hm.wait()

    @pl.when(wid < W_FULL)
    def _():
        pltpu.sync_copy(y_hbm.at[pl.ds(rbase + 3000, 200)],
                        yv.at[pl.ds(3000, 200)])
        pltpu.sync_copy(m_hbm.at[pl.ds(rbase + 3000, 200)],
                        mv.at[pl.ds(3000, 200)])

    @plsc.parallel_loop(0, SPAN // L, unroll=2)
    def _(j):
        sl = pl.ds(j * L, L)
        idxf[sl] = jnp.where(mv[sl] != 0, yv[sl], jnp.int32(NUM_CLASSES))

    h_tab.wait()
    plsc.subcore_barrier()

    gh = {0: issue_g(0)}
    out_flight = {}
    for t in range(T):
        xv, rv, _, _ = bufs[t % 2]
        if t + 1 < T:
            if t >= 1:
                out_flight.pop(t - 1).wait()
                xh[t + 1] = issue_x(t + 1)
            gh[t + 1] = issue_g(t + 1)
        xh.pop(t).wait()
        h0, h1 = gh.pop(t)
        h0.wait()
        h1.wait()

        @plsc.parallel_loop(0, B, unroll=4)
        def _(r):
            for cc in range(D // L):
                sl = pl.ds(cc * L, L)
                xv[r, sl] = xv[r, sl] + rv[r, sl]

        _, base = chunk_base(t)
        out_flight[t] = pltpu.async_copy(xv, out_hbm.at[pl.ds(base, B)],
                                         out_sems[t % 2])
    out_flight.pop(T - 2).wait()
    out_flight.pop(T - 1).wait()


@jax.jit
def _run(x, y, m_i32, table):
    mesh = plsc.VectorSubcoreMesh(core_axis_name="c", subcore_axis_name="s")
    f = functools.partial(
        pl.kernel,
        out_type=jax.ShapeDtypeStruct((N, D), jnp.float32),
        mesh=mesh,
        scratch_types=[
            pltpu.VMEM((SPAN,), jnp.int32),
            pltpu.VMEM((SPAN,), jnp.int32),
            pltpu.VMEM((SPAN,), jnp.int32),
            pltpu.VMEM((B, D), jnp.float32),
            pltpu.VMEM((B, D), jnp.float32),
            pltpu.VMEM((B, D), jnp.float32),
            pltpu.VMEM((B, D), jnp.float32),
            pltpu.VMEM_SHARED((1024, D), jnp.float32),
            pltpu.SemaphoreType.DMA,
            pltpu.SemaphoreType.DMA,
            pltpu.SemaphoreType.DMA,
            pltpu.SemaphoreType.DMA,
            pltpu.SemaphoreType.DMA,
            pltpu.SemaphoreType.DMA,
            pltpu.SemaphoreType.DMA,
        ],
    )(_mask_label_sc)
    return f(x, y, m_i32, table)


def kernel(x, y, mask, emb_weight):
    m_i32 = mask.astype(jnp.int32)
    table = jnp.concatenate(
        [emb_weight, jnp.zeros((24, D), jnp.float32)], axis=0)
    return _run(x, y, m_i32, table)


# speedup vs baseline: 1.0493x; 1.0027x over previous
import functools

import jax
import jax.numpy as jnp
from jax import lax
from jax.experimental import pallas as pl
from jax.experimental.pallas import tpu as pltpu
from jax.experimental.pallas import tpu_sc as plsc

N = 100000
D = 128
NUM_CLASSES = 1000

B = 200
G = 128
T = 16
NW = 32
L = 16
SPAN = B * T
W_FULL = 20


def _mask_label_sc(x_hbm, y_hbm, m_hbm, tab_hbm, out_hbm,
                   yv, mv, idxf, xa, xb, ra, rb, tab_sp,
                   sem_xa, sem_xb, sem_ga, sem_gb, sem_oa, sem_ob, sem_t):
    sid = lax.axis_index("s")
    wid = sid * 2 + lax.axis_index("c")
    cnt = jnp.where(wid < W_FULL, T, T - 1)
    rbase = 3000 * wid + 200 * jnp.minimum(wid, W_FULL)

    bufs = ((xa, ra, sem_xa, sem_ga), (xb, rb, sem_xb, sem_gb))
    out_sems = (sem_oa, sem_ob)

    def chunk_base(t):
        ct = jnp.minimum(jnp.int32(t), cnt - 1)
        return ct, rbase + B * ct

    def issue_x(t):
        xv = bufs[t % 2][0]
        _, base = chunk_base(t)
        return pltpu.async_copy(x_hbm.at[pl.ds(base, B)], xv, bufs[t % 2][2])

    def issue_g(t):
        rv, sg = bufs[t % 2][1], bufs[t % 2][3]
        ct, _ = chunk_base(t)
        ib = ct * B
        h0 = pltpu.async_copy(tab_sp.at[idxf.at[pl.ds(ib, B)]], rv, sg)
        return (h0,)

    h_tab = pltpu.async_copy(tab_hbm.at[pl.ds(64 * sid, 64)],
                             tab_sp.at[pl.ds(64 * sid, 64)], sem_t)
    xh = {0: issue_x(0), 1: issue_x(1)}
    hy = pltpu.async_copy(y_hbm.at[pl.ds(rbase, 3000)],
                          yv.at[pl.ds(0, 3000)], sem_oa)
    hm = pltpu.async_copy(m_hbm.at[pl.ds(rbase, 3000)],
                          mv.at[pl.ds(0, 3000)], sem_ob)
    hy.wait()
    hm.wait()

    @pl.when(wid < W_FULL)
    def _():
        pltpu.sync_copy(y_hbm.at[pl.ds(rbase + 3000, 200)],
                        yv.at[pl.ds(3000, 200)])
        pltpu.sync_copy(m_hbm.at[pl.ds(rbase + 3000, 200)],
                        mv.at[pl.ds(3000, 200)])

    @plsc.parallel_loop(0, SPAN // L, unroll=2)
    def _(j):
        sl = pl.ds(j * L, L)
        idxf[sl] = jnp.where(mv[sl] != 0, yv[sl], jnp.int32(NUM_CLASSES))

    h_tab.wait()
    plsc.subcore_barrier()

    gh = {0: issue_g(0)}
    out_flight = {}
    for t in range(T):
        xv, rv, _, _ = bufs[t % 2]
        if t + 1 < T:
            if t >= 1:
                out_flight.pop(t - 1).wait()
                xh[t + 1] = issue_x(t + 1)
            gh[t + 1] = issue_g(t + 1)
        xh.pop(t).wait()
        (h0,) = gh.pop(t)
        h0.wait()

        @plsc.parallel_loop(0, B, unroll=4)
        def _(r):
            for cc in range(D // L):
                sl = pl.ds(cc * L, L)
                xv[r, sl] = xv[r, sl] + rv[r, sl]

        _, base = chunk_base(t)
        out_flight[t] = pltpu.async_copy(xv, out_hbm.at[pl.ds(base, B)],
                                         out_sems[t % 2])
    out_flight.pop(T - 2).wait()
    out_flight.pop(T - 1).wait()


@jax.jit
def _run(x, y, m_i32, table):
    mesh = plsc.VectorSubcoreMesh(core_axis_name="c", subcore_axis_name="s")
    f = functools.partial(
        pl.kernel,
        out_type=jax.ShapeDtypeStruct((N, D), jnp.float32),
        mesh=mesh,
        scratch_types=[
            pltpu.VMEM((SPAN,), jnp.int32),
            pltpu.VMEM((SPAN,), jnp.int32),
            pltpu.VMEM((SPAN,), jnp.int32),
            pltpu.VMEM((B, D), jnp.float32),
            pltpu.VMEM((B, D), jnp.float32),
            pltpu.VMEM((B, D), jnp.float32),
            pltpu.VMEM((B, D), jnp.float32),
            pltpu.VMEM_SHARED((1024, D), jnp.float32),
            pltpu.SemaphoreType.DMA,
            pltpu.SemaphoreType.DMA,
            pltpu.SemaphoreType.DMA,
            pltpu.SemaphoreType.DMA,
            pltpu.SemaphoreType.DMA,
            pltpu.SemaphoreType.DMA,
            pltpu.SemaphoreType.DMA,
        ],
    )(_mask_label_sc)
    return f(x, y, m_i32, table)


def kernel(x, y, mask, emb_weight):
    m_i32 = mask.astype(jnp.int32)
    table = jnp.concatenate(
        [emb_weight, jnp.zeros((24, D), jnp.float32)], axis=0)
    return _run(x, y, m_i32, table)
